# baseline (speedup 1.0000x reference)
import functools

import jax
import jax.numpy as jnp
from jax import lax
from jax.experimental import pallas as pl
from jax.experimental.pallas import tpu as pltpu
from jax.experimental.pallas import tpu_sc as plsc

N = 10000
D = 128
E = 320000
NC = 2
NS = 16
NW = NC * NS
BE = 128
NB = 80
NBC = 16
EPW = NB * BE
E_PAD = EPW * NW
N_PAD = 10240
RPT = N_PAD // NS
DH = D // 2

LSTM_DIM = 128
B = 4
T = 10
NPER = 250
EDGETYPE = 1
FC1 = 64
FC2 = 8

_MESH = plsc.VectorSubcoreMesh(core_axis_name="c", subcore_axis_name="s")


@functools.partial(
    pl.kernel,
    out_type=jax.ShapeDtypeStruct((NC, N_PAD), jnp.float32),
    mesh=_MESH,
    scratch_types=[
        pltpu.VMEM((NB, BE), jnp.int32),
        pltpu.VMEM((NB, BE), jnp.float32),
        pltpu.VMEM((RPT,), jnp.float32),
        pltpu.VMEM_SHARED((N_PAD,), jnp.float32),
    ],
)
def _sc_deg(col_hbm, ew_hbm, zrow_hbm, out_hbm, col_v, ew_v, z_v, acc):
    cid = lax.axis_index("c")
    sid = lax.axis_index("s")
    wid = sid * NC + cid
    pltpu.sync_copy(col_hbm.at[pl.ds(wid * NB, NB)], col_v)
    pltpu.sync_copy(ew_hbm.at[pl.ds(wid * NB, NB)], ew_v)
    pltpu.sync_copy(zrow_hbm, z_v)
    pltpu.sync_copy(z_v, acc.at[pl.ds(sid * RPT, RPT)])
    plsc.subcore_barrier()

    def body(j, carry):
        pltpu.sync_copy(ew_v.at[j], acc.at[col_v.at[j]], add=True)
        return carry

    lax.fori_loop(0, NB, body, 0)
    plsc.subcore_barrier()
    pltpu.sync_copy(acc.at[pl.ds(sid * RPT, RPT)], z_v)
    pltpu.sync_copy(z_v, out_hbm.at[cid, pl.ds(sid * RPT, RPT)])


@functools.partial(
    pl.kernel,
    out_type=jax.ShapeDtypeStruct((E_PAD, D), jnp.float32),
    mesh=_MESH,
    scratch_types=[
        pltpu.VMEM((NBC, BE), jnp.int32),
        pltpu.VMEM((NBC, BE), jnp.float32),
        pltpu.VMEM((BE, D), jnp.float32),
        pltpu.VMEM((BE, D), jnp.float32),
        pltpu.VMEM_SHARED((N_PAD, D), jnp.float32),
        pltpu.SemaphoreType.DMA,
        pltpu.SemaphoreType.DMA,
    ],
)
def _sc_msg(y_hbm, row_hbm, ew_hbm, msg_hbm, row_v, ew_v, rows0, rows1, ysh,
            sem0, sem1):
    cid = lax.axis_index("c")
    sid = lax.axis_index("s")
    wid = sid * NC + cid
    for k in range(RPT // BE):
        r0 = sid * RPT + k * BE
        pltpu.sync_copy(y_hbm.at[pl.ds(r0, BE)], rows0)
        pltpu.sync_copy(rows0, ysh.at[pl.ds(r0, BE)])
    plsc.subcore_barrier()

    def proc(cc, jj, rows_v):
        def scale(g, c3):
            gbase = pl.multiple_of(g * 16, 16)
            wvec = ew_v[jj, pl.ds(gbase, 16)]
            for lane in range(16):
                e = gbase + lane
                w = jnp.broadcast_to(wvec[lane], (16,))
                for k in range(D // 16):
                    rows_v[e, pl.ds(k * 16, 16)] = rows_v[e, pl.ds(k * 16, 16)] * w
            return c3

        lax.fori_loop(0, BE // 16, scale, 0)
        pltpu.sync_copy(
            rows_v, msg_hbm.at[pl.ds((wid * NB + cc * NBC + jj) * BE, BE)])

    def chunk(cc, carry):
        pltpu.sync_copy(row_hbm.at[pl.ds(wid * NB + cc * NBC, NBC)], row_v)
        pltpu.sync_copy(ew_hbm.at[pl.ds(wid * NB + cc * NBC, NBC)], ew_v)
        pltpu.async_copy(ysh.at[row_v.at[0]], rows0, sem0)

        def body(jj, c2):
            j0 = jj * 2
            pltpu.async_copy(ysh.at[row_v.at[j0 + 1]], rows1, sem1)
            pltpu.make_async_copy(ysh.at[row_v.at[j0]], rows0, sem0).wait()
            proc(cc, j0, rows0)

            @pl.when(jj < NBC // 2 - 1)
            def _():
                pltpu.async_copy(ysh.at[row_v.at[j0 + 2]], rows0, sem0)

            pltpu.make_async_copy(ysh.at[row_v.at[j0 + 1]], rows1, sem1).wait()
            proc(cc, j0 + 1, rows1)
            return c2

        lax.fori_loop(0, NBC // 2, body, 0)
        return carry

    lax.fori_loop(0, NB // NBC, chunk, 0)


@functools.partial(
    pl.kernel,
    out_type=jax.ShapeDtypeStruct((NC, N_PAD, D), jnp.float32),
    mesh=_MESH,
    scratch_types=[
        pltpu.VMEM((NBC, BE), jnp.int32),
        pltpu.VMEM((BE, D), jnp.float32),
        pltpu.VMEM((BE, D), jnp.float32),
        pltpu.VMEM_SHARED((N_PAD, D), jnp.float32),
        pltpu.SemaphoreType.DMA,
        pltpu.SemaphoreType.DMA,
        pltpu.SemaphoreType.DMA,
        pltpu.SemaphoreType.DMA,
    ],
)
def _sc_scat(msg_hbm, col_hbm, zblk_hbm, out_hbm,
             col_v, buf0, buf1, acc, sem0, sem1, semw0, semw1):
    cid = lax.axis_index("c")
    sid = lax.axis_index("s")
    wid = sid * NC + cid
    pltpu.sync_copy(zblk_hbm, buf0)
    for k in range(RPT // BE):
        pltpu.sync_copy(buf0, acc.at[pl.ds(sid * RPT + k * BE, BE)])
    plsc.subcore_barrier()

    def chunk(cc, carry):
        pltpu.sync_copy(col_hbm.at[pl.ds(wid * NB + cc * NBC, NBC)], col_v)
        base = (wid * NB + cc * NBC) * BE
        pltpu.async_copy(msg_hbm.at[pl.ds(base, BE)], buf0, sem0)

        def body(jj, c2):
            j0 = jj * 2
            pltpu.make_async_copy(msg_hbm.at[pl.ds(base + j0 * BE, BE)], buf0, sem0).wait()

            @pl.when(jj > 0)
            def _():
                pltpu.make_async_copy(buf1, acc.at[col_v.at[j0]], semw1).wait()

            pltpu.async_copy(msg_hbm.at[pl.ds(base + (j0 + 1) * BE, BE)], buf1, sem1)
            pltpu.async_copy(buf0, acc.at[col_v.at[j0]], semw0, add=True)
            pltpu.make_async_copy(msg_hbm.at[pl.ds(base + (j0 + 1) * BE, BE)], buf1, sem1).wait()

            @pl.when(jj < NBC // 2 - 1)
            def _():
                pltpu.make_async_copy(buf0, acc.at[col_v.at[j0]], semw0).wait()
                pltpu.async_copy(msg_hbm.at[pl.ds(base + (j0 + 2) * BE, BE)], buf0, sem0)

            pltpu.async_copy(buf1, acc.at[col_v.at[j0 + 1]], semw1, add=True)
            return c2

        lax.fori_loop(0, NBC // 2, body, 0)
        pltpu.make_async_copy(buf0, acc.at[col_v.at[0]], semw0).wait()
        pltpu.make_async_copy(buf1, acc.at[col_v.at[0]], semw1).wait()
        return carry

    lax.fori_loop(0, NB // NBC, chunk, 0)
    plsc.subcore_barrier()
    for k in range(RPT // BE):
        r0 = sid * RPT + k * BE
        pltpu.sync_copy(acc.at[pl.ds(r0, BE)], buf0)
        pltpu.sync_copy(buf0, out_hbm.at[cid, pl.ds(r0, BE)])


NSEQ = B * NPER
H = LSTM_DIM


def _tc_prep_k(x_ref, w_ref, o_ref):
    x = x_ref[...]
    mu = jnp.mean(x, axis=0, keepdims=True)
    var = jnp.sum((x - mu) ** 2, axis=0, keepdims=True) / (x.shape[0] - 1)
    xn = (x - mu) * lax.rsqrt(var)
    o_ref[...] = jnp.dot(xn, w_ref[...], preferred_element_type=jnp.float32)


def _tc_prep(x, w):
    return pl.pallas_call(
        _tc_prep_k, out_shape=jax.ShapeDtypeStruct((N, D), jnp.float32))(x, w)


def _dis_col(degt_ref):
    deg = degt_ref[:, 0:1] + degt_ref[:, 1:2] + 1.0
    return lax.rsqrt(deg)


def _tc_scale_k(degt_ref, xw_ref, o_ref):
    o_ref[...] = _dis_col(degt_ref) * xw_ref[...]


def _tc_scale(degt, xwp):
    return pl.pallas_call(
        _tc_scale_k,
        out_shape=jax.ShapeDtypeStruct((N_PAD, D), jnp.float32))(degt, xwp)


def _tc_comb_mm_k(sp_ref, yp_ref, degt_ref, b_ref, w_ref, o_ref):
    dis = _dis_col(degt_ref)
    h = jax.nn.relu(dis * (sp_ref[0] + sp_ref[1] + yp_ref[...]) + b_ref[...])
    o_ref[...] = dis * jnp.dot(h, w_ref[...], preferred_element_type=jnp.float32)


def _tc_comb_mm(sp, yp, degt, b, w):
    return pl.pallas_call(
        _tc_comb_mm_k,
        out_shape=jax.ShapeDtypeStruct((N_PAD, D), jnp.float32),
    )(sp, yp, degt, b.reshape(1, D), w)


def _tc_comb_last_k(sp_ref, yp_ref, degt_ref, b_ref, o_ref):
    dis = _dis_col(degt_ref)
    o_ref[...] = jax.nn.relu(
        dis * (sp_ref[0] + sp_ref[1] + yp_ref[...]) + b_ref[...])


def _tc_comb_last(sp, yp, degt, b):
    return pl.pallas_call(
        _tc_comb_last_k,
        out_shape=jax.ShapeDtypeStruct((N_PAD, D), jnp.float32),
    )(sp, yp, degt, b.reshape(1, D))


def _tc_lstm_head_k(ts_ref, wih_ref, whh_ref, bb_ref, wf1_ref, bf1_ref,
                    wf2_ref, bf2_ref, o_ref, h_ref, c_ref):
    t = pl.program_id(0)

    @pl.when(t == 0)
    def _():
        h_ref[...] = jnp.zeros((NSEQ, H), jnp.float32)
        c_ref[...] = jnp.zeros((NSEQ, H), jnp.float32)

    xt = ts_ref[0]
    gates = (jnp.dot(xt, wih_ref[...].T, preferred_element_type=jnp.float32)
             + jnp.dot(h_ref[...], whh_ref[...].T,
                       preferred_element_type=jnp.float32)
             + bb_ref[...])
    i = jax.nn.sigmoid(gates[:, 0:H])
    f = jax.nn.sigmoid(gates[:, H:2 * H])
    g = jnp.tanh(gates[:, 2 * H:3 * H])
    o = jax.nn.sigmoid(gates[:, 3 * H:4 * H])
    c = f * c_ref[...] + i * g
    h = o * jnp.tanh(c)
    c_ref[...] = c
    h_ref[...] = h

    @pl.when(t == T - 1)
    def _():
        fc1 = jax.nn.relu(
            jnp.dot(h, wf1_ref[...].T, preferred_element_type=jnp.float32)
            + bf1_ref[...])
        logits = (jnp.dot(fc1, wf2_ref[...].T,
                          preferred_element_type=jnp.float32) + bf2_ref[...])
        o_ref[...] = jax.nn.softmax(logits, axis=1)


def _tc_lstm_head(ts, Wih, Whh, bih, bhh, Wf1, bf1, Wf2, bf2):
    return pl.pallas_call(
        _tc_lstm_head_k,
        grid=(T,),
        in_specs=[
            pl.BlockSpec((1, NSEQ, D), lambda t: (t, 0, 0)),
            pl.BlockSpec((4 * H, D), lambda t: (0, 0)),
            pl.BlockSpec((4 * H, H), lambda t: (0, 0)),
            pl.BlockSpec((1, 4 * H), lambda t: (0, 0)),
            pl.BlockSpec((FC1, H), lambda t: (0, 0)),
            pl.BlockSpec((1, FC1), lambda t: (0, 0)),
            pl.BlockSpec((FC2, FC1), lambda t: (0, 0)),
            pl.BlockSpec((1, FC2), lambda t: (0, 0)),
        ],
        out_specs=pl.BlockSpec((NSEQ, FC2), lambda t: (0, 0)),
        out_shape=jax.ShapeDtypeStruct((NSEQ, FC2), jnp.float32),
        scratch_shapes=[
            pltpu.VMEM((NSEQ, H), jnp.float32),
            pltpu.VMEM((NSEQ, H), jnp.float32),
        ],
    )(ts, Wih, Whh, (bih + bhh).reshape(1, 4 * H), Wf1, bf1.reshape(1, FC1),
      Wf2, bf2.reshape(1, FC2))


def kernel(x, edge_index, edge_attr, batch, seq, Wih, Whh, bih, bhh,
           W1, b1, W2, b2, Wf1, bf1, Wf2, bf2):
    ew = jnp.abs(edge_attr[:, EDGETYPE])
    row = edge_index[0]
    col = edge_index[1]

    pad = E_PAD - E
    row_p = jnp.concatenate([row, jnp.zeros((pad,), row.dtype)]).reshape(NW * NB, BE)
    col_p = jnp.concatenate([col, jnp.zeros((pad,), col.dtype)]).reshape(NW * NB, BE)
    ew_p = jnp.concatenate([ew, jnp.zeros((pad,), ew.dtype)]).reshape(NW * NB, BE)

    zrow = jnp.zeros((RPT,), jnp.float32)
    zblk = jnp.zeros((BE, D), jnp.float32)

    xw1 = _tc_prep(x, W1)
    degp = _sc_deg(col_p, ew_p, zrow)
    degt = jnp.swapaxes(degp, 0, 1)

    def conv_sc(yp):
        msg = _sc_msg(yp, row_p, ew_p)
        return _sc_scat(msg, col_p, zblk)

    y1p = _tc_scale(degt, jnp.pad(xw1, ((0, N_PAD - N), (0, 0))))
    sp1 = conv_sc(y1p)
    y2p = _tc_comb_mm(sp1, y1p, degt, b1, W2)
    sp2 = conv_sc(y2p)
    h2p = _tc_comb_last(sp2, y2p, degt, b2)

    t3 = h2p[:N].reshape(B, T, NPER, LSTM_DIM)
    ts = jnp.transpose(t3, (1, 0, 2, 3)).reshape(T, -1, LSTM_DIM)
    out = _tc_lstm_head(ts, Wih, Whh, bih, bhh, Wf1, bf1, Wf2, bf2)
    return out.reshape(B, -1, FC2)


# speedup vs baseline: 1.0522x; 1.0522x over previous
import functools

import jax
import jax.numpy as jnp
from jax import lax
from jax.experimental import pallas as pl
from jax.experimental.pallas import tpu as pltpu
from jax.experimental.pallas import tpu_sc as plsc

N = 10000
D = 128
E = 320000
NC = 2
NS = 16
NW = NC * NS
BE = 128
NB = 80
NBC = 16
EPW = NB * BE
E_PAD = EPW * NW
N_PAD = 10240
RPT = N_PAD // NS
DH = D // 2

LSTM_DIM = 128
B = 4
T = 10
NPER = 250
EDGETYPE = 1
FC1 = 64
FC2 = 8

_MESH = plsc.VectorSubcoreMesh(core_axis_name="c", subcore_axis_name="s")


@functools.partial(
    pl.kernel,
    out_type=jax.ShapeDtypeStruct((NC, N_PAD), jnp.float32),
    mesh=_MESH,
    scratch_types=[
        pltpu.VMEM((NB, BE), jnp.int32),
        pltpu.VMEM((NB, BE), jnp.float32),
        pltpu.VMEM((RPT,), jnp.float32),
        pltpu.VMEM_SHARED((N_PAD,), jnp.float32),
    ],
)
def _sc_deg(col_hbm, ew_hbm, zrow_hbm, out_hbm, col_v, ew_v, z_v, acc):
    cid = lax.axis_index("c")
    sid = lax.axis_index("s")
    wid = sid * NC + cid
    pltpu.sync_copy(col_hbm.at[pl.ds(wid * NB, NB)], col_v)
    pltpu.sync_copy(ew_hbm.at[pl.ds(wid * NB, NB)], ew_v)
    pltpu.sync_copy(zrow_hbm, z_v)
    pltpu.sync_copy(z_v, acc.at[pl.ds(sid * RPT, RPT)])
    plsc.subcore_barrier()

    def body(j, carry):
        pltpu.sync_copy(ew_v.at[j], acc.at[col_v.at[j]], add=True)
        return carry

    lax.fori_loop(0, NB, body, 0)
    plsc.subcore_barrier()
    pltpu.sync_copy(acc.at[pl.ds(sid * RPT, RPT)], z_v)
    pltpu.sync_copy(z_v, out_hbm.at[cid, pl.ds(sid * RPT, RPT)])


@functools.partial(
    pl.kernel,
    out_type=jax.ShapeDtypeStruct((E_PAD, D), jnp.float32),
    mesh=_MESH,
    scratch_types=[
        pltpu.VMEM((NBC, BE), jnp.int32),
        pltpu.VMEM((NBC, BE), jnp.float32),
        pltpu.VMEM((BE, D), jnp.float32),
        pltpu.VMEM((BE, D), jnp.float32),
        pltpu.VMEM_SHARED((N_PAD, D), jnp.float32),
        pltpu.SemaphoreType.DMA,
        pltpu.SemaphoreType.DMA,
    ],
)
def _sc_msg(y_hbm, row_hbm, ew_hbm, msg_hbm, row_v, ew_v, rows0, rows1, ysh,
            sem0, sem1):
    cid = lax.axis_index("c")
    sid = lax.axis_index("s")
    wid = sid * NC + cid
    for k in range(RPT // BE):
        r0 = sid * RPT + k * BE
        pltpu.sync_copy(y_hbm.at[pl.ds(r0, BE)], rows0)
        pltpu.sync_copy(rows0, ysh.at[pl.ds(r0, BE)])
    plsc.subcore_barrier()

    def proc(cc, jj, rows_v):
        def scale(g, c3):
            gbase = pl.multiple_of(g * 16, 16)
            wvec = ew_v[jj, pl.ds(gbase, 16)]
            for lane in range(16):
                e = gbase + lane
                w = jnp.broadcast_to(wvec[lane], (16,))
                for k in range(D // 16):
                    rows_v[e, pl.ds(k * 16, 16)] = rows_v[e, pl.ds(k * 16, 16)] * w
            return c3

        lax.fori_loop(0, BE // 16, scale, 0)
        pltpu.sync_copy(
            rows_v, msg_hbm.at[pl.ds((wid * NB + cc * NBC + jj) * BE, BE)])

    def chunk(cc, carry):
        pltpu.sync_copy(row_hbm.at[pl.ds(wid * NB + cc * NBC, NBC)], row_v)
        pltpu.sync_copy(ew_hbm.at[pl.ds(wid * NB + cc * NBC, NBC)], ew_v)
        pltpu.async_copy(ysh.at[row_v.at[0]], rows0, sem0)

        def body(jj, c2):
            j0 = jj * 2
            pltpu.async_copy(ysh.at[row_v.at[j0 + 1]], rows1, sem1)
            pltpu.make_async_copy(ysh.at[row_v.at[j0]], rows0, sem0).wait()
            proc(cc, j0, rows0)

            @pl.when(jj < NBC // 2 - 1)
            def _():
                pltpu.async_copy(ysh.at[row_v.at[j0 + 2]], rows0, sem0)

            pltpu.make_async_copy(ysh.at[row_v.at[j0 + 1]], rows1, sem1).wait()
            proc(cc, j0 + 1, rows1)
            return c2

        lax.fori_loop(0, NBC // 2, body, 0)
        return carry

    lax.fori_loop(0, NB // NBC, chunk, 0)


@functools.partial(
    pl.kernel,
    out_type=jax.ShapeDtypeStruct((NC, N_PAD, D), jnp.float32),
    mesh=_MESH,
    scratch_types=[
        pltpu.VMEM((NBC, BE), jnp.int32),
        pltpu.VMEM((BE, D), jnp.float32),
        pltpu.VMEM((BE, D), jnp.float32),
        pltpu.VMEM_SHARED((N_PAD, D), jnp.float32),
        pltpu.SemaphoreType.DMA,
        pltpu.SemaphoreType.DMA,
    ],
)
def _sc_scat(msg_hbm, col_hbm, zblk_hbm, out_hbm,
             col_v, buf0, buf1, acc, sem0, sem1):
    cid = lax.axis_index("c")
    sid = lax.axis_index("s")
    wid = sid * NC + cid
    pltpu.sync_copy(zblk_hbm, buf0)
    for k in range(RPT // BE):
        pltpu.sync_copy(buf0, acc.at[pl.ds(sid * RPT + k * BE, BE)])
    plsc.subcore_barrier()

    def chunk(cc, carry):
        pltpu.sync_copy(col_hbm.at[pl.ds(wid * NB + cc * NBC, NBC)], col_v)
        base = (wid * NB + cc * NBC) * BE
        pltpu.async_copy(msg_hbm.at[pl.ds(base, BE)], buf0, sem0)

        def body(jj, c2):
            j0 = jj * 2
            pltpu.async_copy(msg_hbm.at[pl.ds(base + (j0 + 1) * BE, BE)], buf1, sem1)
            pltpu.make_async_copy(msg_hbm.at[pl.ds(base + j0 * BE, BE)], buf0, sem0).wait()
            pltpu.sync_copy(buf0, acc.at[col_v.at[j0]], add=True)

            @pl.when(jj < NBC // 2 - 1)
            def _():
                pltpu.async_copy(msg_hbm.at[pl.ds(base + (j0 + 2) * BE, BE)], buf0, sem0)

            pltpu.make_async_copy(msg_hbm.at[pl.ds(base + (j0 + 1) * BE, BE)], buf1, sem1).wait()
            pltpu.sync_copy(buf1, acc.at[col_v.at[j0 + 1]], add=True)
            return c2

        lax.fori_loop(0, NBC // 2, body, 0)
        return carry

    lax.fori_loop(0, NB // NBC, chunk, 0)
    plsc.subcore_barrier()
    for k in range(RPT // BE):
        r0 = sid * RPT + k * BE
        pltpu.sync_copy(acc.at[pl.ds(r0, BE)], buf0)
        pltpu.sync_copy(buf0, out_hbm.at[cid, pl.ds(r0, BE)])


NSEQ = B * NPER
H = LSTM_DIM


def _tc_prep_k(x_ref, w_ref, o_ref):
    x = x_ref[...]
    mu = jnp.mean(x, axis=0, keepdims=True)
    var = jnp.sum((x - mu) ** 2, axis=0, keepdims=True) / (x.shape[0] - 1)
    xn = (x - mu) * lax.rsqrt(var)
    o_ref[...] = jnp.dot(xn, w_ref[...], preferred_element_type=jnp.float32)


def _tc_prep(x, w):
    return pl.pallas_call(
        _tc_prep_k, out_shape=jax.ShapeDtypeStruct((N, D), jnp.float32))(x, w)


def _dis_col(degt_ref):
    deg = degt_ref[:, 0:1] + degt_ref[:, 1:2] + 1.0
    return lax.rsqrt(deg)


def _tc_scale_k(degt_ref, xw_ref, o_ref):
    o_ref[...] = _dis_col(degt_ref) * xw_ref[...]


def _tc_scale(degt, xwp):
    return pl.pallas_call(
        _tc_scale_k,
        out_shape=jax.ShapeDtypeStruct((N_PAD, D), jnp.float32))(degt, xwp)


def _tc_comb_mm_k(sp_ref, yp_ref, degt_ref, b_ref, w_ref, o_ref):
    dis = _dis_col(degt_ref)
    h = jax.nn.relu(dis * (sp_ref[0] + sp_ref[1] + yp_ref[...]) + b_ref[...])
    o_ref[...] = dis * jnp.dot(h, w_ref[...], preferred_element_type=jnp.float32)


def _tc_comb_mm(sp, yp, degt, b, w):
    return pl.pallas_call(
        _tc_comb_mm_k,
        out_shape=jax.ShapeDtypeStruct((N_PAD, D), jnp.float32),
    )(sp, yp, degt, b.reshape(1, D), w)


def _tc_comb_last_k(sp_ref, yp_ref, degt_ref, b_ref, o_ref):
    dis = _dis_col(degt_ref)
    o_ref[...] = jax.nn.relu(
        dis * (sp_ref[0] + sp_ref[1] + yp_ref[...]) + b_ref[...])


def _tc_comb_last(sp, yp, degt, b):
    return pl.pallas_call(
        _tc_comb_last_k,
        out_shape=jax.ShapeDtypeStruct((N_PAD, D), jnp.float32),
    )(sp, yp, degt, b.reshape(1, D))


def _tc_lstm_head_k(ts_ref, wih_ref, whh_ref, bb_ref, wf1_ref, bf1_ref,
                    wf2_ref, bf2_ref, o_ref, h_ref, c_ref):
    t = pl.program_id(0)

    @pl.when(t == 0)
    def _():
        h_ref[...] = jnp.zeros((NSEQ, H), jnp.float32)
        c_ref[...] = jnp.zeros((NSEQ, H), jnp.float32)

    xt = ts_ref[0]
    gates = (jnp.dot(xt, wih_ref[...].T, preferred_element_type=jnp.float32)
             + jnp.dot(h_ref[...], whh_ref[...].T,
                       preferred_element_type=jnp.float32)
             + bb_ref[...])
    i = jax.nn.sigmoid(gates[:, 0:H])
    f = jax.nn.sigmoid(gates[:, H:2 * H])
    g = jnp.tanh(gates[:, 2 * H:3 * H])
    o = jax.nn.sigmoid(gates[:, 3 * H:4 * H])
    c = f * c_ref[...] + i * g
    h = o * jnp.tanh(c)
    c_ref[...] = c
    h_ref[...] = h

    @pl.when(t == T - 1)
    def _():
        fc1 = jax.nn.relu(
            jnp.dot(h, wf1_ref[...].T, preferred_element_type=jnp.float32)
            + bf1_ref[...])
        logits = (jnp.dot(fc1, wf2_ref[...].T,
                          preferred_element_type=jnp.float32) + bf2_ref[...])
        o_ref[...] = jax.nn.softmax(logits, axis=1)


def _tc_lstm_head(ts, Wih, Whh, bih, bhh, Wf1, bf1, Wf2, bf2):
    return pl.pallas_call(
        _tc_lstm_head_k,
        grid=(T,),
        in_specs=[
            pl.BlockSpec((1, NSEQ, D), lambda t: (t, 0, 0)),
            pl.BlockSpec((4 * H, D), lambda t: (0, 0)),
            pl.BlockSpec((4 * H, H), lambda t: (0, 0)),
            pl.BlockSpec((1, 4 * H), lambda t: (0, 0)),
            pl.BlockSpec((FC1, H), lambda t: (0, 0)),
            pl.BlockSpec((1, FC1), lambda t: (0, 0)),
            pl.BlockSpec((FC2, FC1), lambda t: (0, 0)),
            pl.BlockSpec((1, FC2), lambda t: (0, 0)),
        ],
        out_specs=pl.BlockSpec((NSEQ, FC2), lambda t: (0, 0)),
        out_shape=jax.ShapeDtypeStruct((NSEQ, FC2), jnp.float32),
        scratch_shapes=[
            pltpu.VMEM((NSEQ, H), jnp.float32),
            pltpu.VMEM((NSEQ, H), jnp.float32),
        ],
    )(ts, Wih, Whh, (bih + bhh).reshape(1, 4 * H), Wf1, bf1.reshape(1, FC1),
      Wf2, bf2.reshape(1, FC2))


def kernel(x, edge_index, edge_attr, batch, seq, Wih, Whh, bih, bhh,
           W1, b1, W2, b2, Wf1, bf1, Wf2, bf2):
    ew = jnp.abs(edge_attr[:, EDGETYPE])
    row = edge_index[0]
    col = edge_index[1]

    pad = E_PAD - E
    row_p = jnp.concatenate([row, jnp.zeros((pad,), row.dtype)]).reshape(NW * NB, BE)
    col_p = jnp.concatenate([col, jnp.zeros((pad,), col.dtype)]).reshape(NW * NB, BE)
    ew_p = jnp.concatenate([ew, jnp.zeros((pad,), ew.dtype)]).reshape(NW * NB, BE)

    zrow = jnp.zeros((RPT,), jnp.float32)
    zblk = jnp.zeros((BE, D), jnp.float32)

    xw1 = _tc_prep(x, W1)
    degp = _sc_deg(col_p, ew_p, zrow)
    degt = jnp.swapaxes(degp, 0, 1)

    def conv_sc(yp):
        msg = _sc_msg(yp, row_p, ew_p)
        return _sc_scat(msg, col_p, zblk)

    y1p = _tc_scale(degt, jnp.pad(xw1, ((0, N_PAD - N), (0, 0))))
    sp1 = conv_sc(y1p)
    y2p = _tc_comb_mm(sp1, y1p, degt, b1, W2)
    sp2 = conv_sc(y2p)
    h2p = _tc_comb_last(sp2, y2p, degt, b2)

    t3 = h2p[:N].reshape(B, T, NPER, LSTM_DIM)
    ts = jnp.transpose(t3, (1, 0, 2, 3)).reshape(T, -1, LSTM_DIM)
    out = _tc_lstm_head(ts, Wih, Whh, bih, bhh, Wf1, bf1, Wf2, bf2)
    return out.reshape(B, -1, FC2)
